# Initial kernel scaffold; baseline (speedup 1.0000x reference)
import jax
import jax.numpy as jnp
from jax.experimental import pallas as pl


def kernel(x, pa_w, pa_b, ca_w1, ca_w2, sc_w, sc_s, sc_b, pj_w, pj_s, pj_b, pw_w):
    raise NotImplementedError("write your pallas kernel here")


# speedup vs baseline: 1.7728x; 1.7728x over previous
import jax
import jax.numpy as jnp
from jax.experimental import pallas as pl
from jax.experimental.pallas import tpu as pltpu

_VMEM_LIMIT = 62 * 1024 * 1024


def _dw3x3(v, w9_ref, bias4, C, H, W, G):
    z1 = jnp.zeros((C, 1, W), jnp.float32)
    ve = jnp.concatenate([z1, v, z1], axis=1)
    zc = jnp.zeros((C, H + 2, 1), jnp.float32)
    vm = jnp.concatenate([zc, ve[:, :, : W - 1]], axis=2)
    vp = jnp.concatenate([ve[:, :, 1:], zc], axis=2)
    taps = (vm, ve, vp)
    acc = None
    for dy in range(3):
        for dx in range(3):
            win = taps[dx][:, dy:dy + H, :].reshape(C, G, 8, W)
            term = win * w9_ref[dy * 3 + dx][:, None]
            acc = term if acc is None else acc + term
    if bias4 is not None:
        acc = acc + bias4
    return acc


def _frh_kernel(x_ref, paw_ref, pab_ref, pjw_ref, caw1_ref, caw2_ref,
                pww_ref, scw_ref, bias_ref, o_ref):
    x = x_ref[0]
    C, H, W = x.shape
    G = H // 8
    xg = x.reshape(C, G, 8, W)

    s1 = jnp.sum(xg, axis=1)
    s2 = jnp.sum(s1, axis=1, keepdims=True)
    pooled = jnp.sum(s2, axis=2, keepdims=True) * (1.0 / (H * W))
    t = caw1_ref[...] * pooled[:, 0]
    h1 = jnp.clip(jnp.sum(t, axis=0, keepdims=True), 0.0, 6.0)
    g = jnp.sum(caw2_ref[...] * h1, axis=1, keepdims=True)
    gate = jax.nn.sigmoid(g)[:, None, :, None]

    pa = _dw3x3(x, paw_ref, pab_ref[...][:, None], C, H, W, G)
    y = xg * (jax.nn.sigmoid(pa) + gate)

    pj = _dw3x3(y.reshape(C, H, W), pjw_ref, None, C, H, W, G).reshape(C, H, W)

    dn = (((1,), (0,)), ((), ()))
    out = (jax.lax.dot_general(pww_ref[...], pj, dn,
                               preferred_element_type=jnp.float32)
           + jax.lax.dot_general(scw_ref[...], x, dn,
                                 preferred_element_type=jnp.float32))
    out4 = out.reshape(C, G, 8, W) + bias_ref[...][:, None]
    o_ref[0] = jnp.clip(out4, 0.0, 6.0).reshape(C, H, W)


def kernel(x, pa_w, pa_b, ca_w1, ca_w2, sc_w, sc_s, sc_b, pj_w, pj_s, pj_b, pw_w):
    B, C, H, W = x.shape
    x = x.astype(jnp.float32)

    bc = lambda a: jnp.broadcast_to(a[:, :, None, None], (a.shape[0], C, 8, W))
    paw_b = bc(pa_w)
    pjw_b = bc(pj_w)
    pab_b = jnp.broadcast_to(pa_b.reshape(C, 1, 1), (C, 8, W))

    pww_t = (pj_s[0][:, None] * pw_w).T
    scw_t = (sc_w * sc_s).T
    bias = (pj_b @ pw_w + sc_b).reshape(C)
    bias_b = jnp.broadcast_to(bias[:, None, None], (C, 8, W))

    Cr = ca_w1.shape[1]
    crp = max(128, ((Cr + 127) // 128) * 128)
    caw1_p = jnp.zeros((C, crp), jnp.float32).at[:, :Cr].set(ca_w1)
    caw2_p = jnp.zeros((C, crp), jnp.float32).at[:, :Cr].set(ca_w2.T)

    def const(a):
        return pl.BlockSpec(a.shape, lambda b: (0,) * a.ndim)

    out = pl.pallas_call(
        _frh_kernel,
        out_shape=jax.ShapeDtypeStruct((B, C, H, W), jnp.float32),
        grid=(B,),
        in_specs=[
            pl.BlockSpec((1, C, H, W), lambda b: (b, 0, 0, 0)),
            const(paw_b), const(pab_b), const(pjw_b),
            const(caw1_p), const(caw2_p),
            const(pww_t), const(scw_t), const(bias_b),
        ],
        out_specs=pl.BlockSpec((1, C, H, W), lambda b: (b, 0, 0, 0)),
        compiler_params=pltpu.CompilerParams(
            dimension_semantics=("parallel",),
            vmem_limit_bytes=_VMEM_LIMIT),
    )(x, paw_b, pab_b, pjw_b, caw1_p, caw2_p, pww_t, scw_t, bias_b)
    return out
